# Initial kernel scaffold; baseline (speedup 1.0000x reference)
import jax
import jax.numpy as jnp
from jax.experimental import pallas as pl


def kernel(nodes, neighs, freq_e, dist_e, feature_labels, spectral3, distance, features_table, weight1, bias1, weight2, bias2):
    raise NotImplementedError("write your pallas kernel here")


# speedup vs baseline: 3.4540x; 3.4540x over previous
import dataclasses
import functools

import jax
import jax.numpy as jnp
from jax import lax
from jax.experimental import pallas as pl
from jax.experimental.pallas import tpu as pltpu
from jax.experimental.pallas import tpu_sc as plsc

NC, NS = 2, 16
NW = NC * NS
CA = 2048
CB = 128
MLP_ROWS = 32
RED_ROWS = 4000


def _sc_mesh():
    return plsc.VectorSubcoreMesh(
        core_axis_name="c", subcore_axis_name="s", num_cores=NC, num_subcores=NS
    )


def _sc_compiler_params():
    cp = pltpu.CompilerParams()
    if "needs_layout_passes" in pltpu.CompilerParams.__dataclass_fields__:
        cp = dataclasses.replace(cp, needs_layout_passes=False)
    return cp


def _small_gather(nodes_rep, neighs_flat, feature_labels, spectral3, distance):
    ppad = nodes_rep.shape[0]
    pw = ppad // NW
    n = feature_labels.shape[0]
    l = distance.shape[0]

    @functools.partial(
        pl.kernel,
        out_type=(
            jax.ShapeDtypeStruct((ppad,), jnp.float32),
            jax.ShapeDtypeStruct((ppad,), jnp.float32),
        ),
        mesh=_sc_mesh(),
        scratch_types=[
            pltpu.VMEM((n,), jnp.int32),
            pltpu.VMEM((n,), jnp.float32),
            pltpu.VMEM((l, l), jnp.float32),
            pltpu.VMEM((CA,), jnp.int32),
            pltpu.VMEM((CA,), jnp.int32),
            pltpu.VMEM((CA,), jnp.float32),
            pltpu.VMEM((CA,), jnp.float32),
        ],
        compiler_params=_sc_compiler_params(),
    )
    def k(nd_hbm, nb_hbm, fl_hbm, s3_hbm, di_hbm, spec_hbm, dcs_hbm,
          fl_v, s3_v, di_v, nd_c, nb_c, sp_c, dc_c):
        wid = lax.axis_index("s") * NC + lax.axis_index("c")
        base = wid * pw
        pltpu.sync_copy(fl_hbm, fl_v)
        pltpu.sync_copy(s3_hbm, s3_v)
        pltpu.sync_copy(di_hbm, di_v)

        @pl.loop(0, pw, step=CA)
        def _chunk(c0):
            pltpu.sync_copy(nd_hbm.at[pl.ds(base + c0, CA)], nd_c)
            pltpu.sync_copy(nb_hbm.at[pl.ds(base + c0, CA)], nb_c)

            @pl.loop(0, CA, step=16)
            def _vec(i):
                nd = nd_c[pl.ds(i, 16)]
                nb = nb_c[pl.ds(i, 16)]
                ls = plsc.load_gather(fl_v, [nd])
                ld = plsc.load_gather(fl_v, [nb])
                sp_c[pl.ds(i, 16)] = plsc.load_gather(s3_v, [nb])
                dc_c[pl.ds(i, 16)] = plsc.load_gather(di_v, [ls, ld])

            pltpu.sync_copy(sp_c, spec_hbm.at[pl.ds(base + c0, CA)])
            pltpu.sync_copy(dc_c, dcs_hbm.at[pl.ds(base + c0, CA)])

    return k(nodes_rep, neighs_flat, feature_labels, spectral3, distance)


def _feat_gather(idx2d, table):
    nrows, _ = idx2d.shape
    d = table.shape[1]
    rows_per_w = nrows // NW

    @functools.partial(
        pl.kernel,
        out_type=jax.ShapeDtypeStruct((nrows * CB, d), jnp.float32),
        mesh=_sc_mesh(),
        scratch_types=[
            pltpu.VMEM((rows_per_w, CB), jnp.int32),
            pltpu.VMEM((CB, d), jnp.float32),
            pltpu.VMEM((CB, d), jnp.float32),
            pltpu.SemaphoreType.DMA,
            pltpu.SemaphoreType.DMA,
        ],
    )
    def k(idx_hbm, tab_hbm, out_hbm, idx_v, row0_v, row1_v, sem0, sem1):
        wid = lax.axis_index("s") * NC + lax.axis_index("c")
        rbase = wid * rows_per_w
        pltpu.sync_copy(idx_hbm.at[pl.ds(rbase, rows_per_w)], idx_v)

        @pl.loop(0, rows_per_w, step=2)
        def _pair(j):
            c0 = pltpu.async_copy(tab_hbm.at[idx_v.at[j]], row0_v, sem0)
            c1 = pltpu.async_copy(tab_hbm.at[idx_v.at[j + 1]], row1_v, sem1)
            c0.wait()
            pltpu.sync_copy(row0_v, out_hbm.at[pl.ds((rbase + j) * CB, CB)])
            c1.wait()
            pltpu.sync_copy(row1_v, out_hbm.at[pl.ds((rbase + j + 1) * CB, CB)])

    return k(idx2d, table)


def _mlp_body(xf_ref, xd_ref, xc_ref, xs_ref, w1_ref, b1_ref, w2_ref, c0_ref,
              w_ref, *, h, inv_s):
    xf = xf_ref[...]
    xd = xd_ref[...]
    xc = xc_ref[...]
    xs = xs_ref[...]

    def kstep(k, acc):
        a = (xf * w1_ref[k, 0] + xd * w1_ref[k, 1] + xc * w1_ref[k, 2]
             + xs * w1_ref[k, 3] + b1_ref[k, 0])
        return acc + w2_ref[k, 0] * jnp.tanh(a)

    acc = lax.fori_loop(0, h, kstep, jnp.zeros_like(xf))
    z = 0.25 * acc + 0.5 * c0_ref[0, 0]
    w_ref[...] = (0.5 * inv_s) * jnp.tanh(z) + (0.5 * inv_s)


def _edge_weights(xf, xd, xc, xs, w1h, b1h, w2c, c0, s):
    rtot = xf.shape[0]
    h = w1h.shape[0]
    grid = rtot // MLP_ROWS
    blk = pl.BlockSpec((MLP_ROWS, 128), lambda i: (i, 0))
    smem = pl.BlockSpec(memory_space=pltpu.SMEM)
    return pl.pallas_call(
        functools.partial(_mlp_body, h=h, inv_s=1.0 / s),
        grid=(grid,),
        in_specs=[blk, blk, blk, blk, smem, smem, smem, smem],
        out_specs=blk,
        out_shape=jax.ShapeDtypeStruct((rtot, 128), jnp.float32),
    )(xf, xd, xc, xs, w1h, b1h, w2c, c0)


def _reduce_body(w_ref, g_ref, o_ref, *, s):
    wg = g_ref[...] * w_ref[...]
    o_ref[...] = jnp.sum(wg.reshape(RED_ROWS // s, s, wg.shape[1]), axis=1)


def _weighted_reduce(wcol, gath, b, s, d):
    grid = (b * s) // RED_ROWS
    return pl.pallas_call(
        functools.partial(_reduce_body, s=s),
        grid=(grid,),
        in_specs=[
            pl.BlockSpec((RED_ROWS, 1), lambda i: (i, 0)),
            pl.BlockSpec((RED_ROWS, d), lambda i: (i, 0)),
        ],
        out_specs=pl.BlockSpec((RED_ROWS // s, d), lambda i: (i, 0)),
        out_shape=jax.ShapeDtypeStruct((b, d), jnp.float32),
    )(wcol, gath)


def kernel(nodes, neighs, freq_e, dist_e, feature_labels, spectral3, distance,
           features_table, weight1, bias1, weight2, bias2):
    b, s = neighs.shape
    n, d = features_table.shape
    h = weight1.shape[0]
    p = b * s

    unit = max(NW * CA, NW * CB * 8)
    ppad = ((p + unit - 1) // unit) * unit
    pad = ppad - p

    nodes_rep = jnp.broadcast_to(nodes[:, None], (b, s)).reshape(p)
    nodes_rep = jnp.pad(nodes_rep, (0, pad))
    neighs_flat = jnp.pad(neighs.reshape(p), (0, pad))

    spec, dcs = _small_gather(
        nodes_rep, neighs_flat,
        feature_labels.astype(jnp.int32), spectral3, distance)

    gath = _feat_gather(neighs_flat.reshape(ppad // CB, CB), features_table)

    rtot = ppad // 128
    xf = jnp.pad(freq_e.reshape(p), (0, pad)).reshape(rtot, 128)
    xd = jnp.pad(dist_e.reshape(p), (0, pad)).reshape(rtot, 128)
    xc = dcs.reshape(rtot, 128)
    xs = spec.reshape(rtot, 128)

    w1h = weight1 * 0.5
    b1h = bias1 * 0.5
    w2c = weight2.reshape(h, 1)
    c0 = (0.5 * jnp.sum(weight2) + bias2[0, 0]).reshape(1, 1)

    w4k = _edge_weights(xf, xd, xc, xs, w1h, b1h, w2c, c0, s)
    wcol = w4k.reshape(ppad)[:p].reshape(p, 1)

    return _weighted_reduce(wcol, gath, b, s, d)


# speedup vs baseline: 3.4544x; 1.0001x over previous
import dataclasses
import functools

import jax
import jax.numpy as jnp
from jax import lax
from jax.experimental import pallas as pl
from jax.experimental.pallas import tpu as pltpu
from jax.experimental.pallas import tpu_sc as plsc

NC, NS = 2, 16
NW = NC * NS
CA = 2048
CB = 128
MLP_ROWS = 32
RED_ROWS = 4000


def _sc_mesh():
    return plsc.VectorSubcoreMesh(
        core_axis_name="c", subcore_axis_name="s", num_cores=NC, num_subcores=NS
    )


def _sc_compiler_params():
    cp = pltpu.CompilerParams()
    if "needs_layout_passes" in pltpu.CompilerParams.__dataclass_fields__:
        cp = dataclasses.replace(cp, needs_layout_passes=False)
    return cp


def _small_gather(nodes_rep, neighs_flat, feature_labels, spectral3, distance):
    ppad = nodes_rep.shape[0]
    pw = ppad // NW
    n = feature_labels.shape[0]
    l = distance.shape[0]

    @functools.partial(
        pl.kernel,
        out_type=(
            jax.ShapeDtypeStruct((ppad,), jnp.float32),
            jax.ShapeDtypeStruct((ppad,), jnp.float32),
        ),
        mesh=_sc_mesh(),
        scratch_types=[
            pltpu.VMEM((n,), jnp.int32),
            pltpu.VMEM((n,), jnp.float32),
            pltpu.VMEM((l, l), jnp.float32),
            pltpu.VMEM((CA,), jnp.int32),
            pltpu.VMEM((CA,), jnp.int32),
            pltpu.VMEM((CA,), jnp.float32),
            pltpu.VMEM((CA,), jnp.float32),
        ],
        compiler_params=_sc_compiler_params(),
    )
    def k(nd_hbm, nb_hbm, fl_hbm, s3_hbm, di_hbm, spec_hbm, dcs_hbm,
          fl_v, s3_v, di_v, nd_c, nb_c, sp_c, dc_c):
        wid = lax.axis_index("s") * NC + lax.axis_index("c")
        base = wid * pw
        pltpu.sync_copy(fl_hbm, fl_v)
        pltpu.sync_copy(s3_hbm, s3_v)
        pltpu.sync_copy(di_hbm, di_v)

        @pl.loop(0, pw, step=CA)
        def _chunk(c0):
            pltpu.sync_copy(nd_hbm.at[pl.ds(base + c0, CA)], nd_c)
            pltpu.sync_copy(nb_hbm.at[pl.ds(base + c0, CA)], nb_c)

            @pl.loop(0, CA, step=16)
            def _vec(i):
                nd = nd_c[pl.ds(i, 16)]
                nb = nb_c[pl.ds(i, 16)]
                ls = plsc.load_gather(fl_v, [nd])
                ld = plsc.load_gather(fl_v, [nb])
                sp_c[pl.ds(i, 16)] = plsc.load_gather(s3_v, [nb])
                dc_c[pl.ds(i, 16)] = plsc.load_gather(di_v, [ls, ld])

            pltpu.sync_copy(sp_c, spec_hbm.at[pl.ds(base + c0, CA)])
            pltpu.sync_copy(dc_c, dcs_hbm.at[pl.ds(base + c0, CA)])

    return k(nodes_rep, neighs_flat, feature_labels, spectral3, distance)


def _feat_gather(idx2d, table):
    nrows, _ = idx2d.shape
    d = table.shape[1]
    rows_per_w = nrows // NW

    nbuf = 4

    @functools.partial(
        pl.kernel,
        out_type=jax.ShapeDtypeStruct((nrows * CB, d), jnp.float32),
        mesh=_sc_mesh(),
        scratch_types=[
            pltpu.VMEM((rows_per_w, CB), jnp.int32),
        ] + [pltpu.VMEM((CB, d), jnp.float32) for _ in range(nbuf)] + [
            pltpu.SemaphoreType.DMA((nbuf,)),
            pltpu.SemaphoreType.DMA((nbuf,)),
        ],
    )
    def k(idx_hbm, tab_hbm, out_hbm, idx_v, *rest):
        bufs = rest[:nbuf]
        gs, ws = rest[nbuf], rest[nbuf + 1]
        wid = lax.axis_index("s") * NC + lax.axis_index("c")
        rbase = wid * rows_per_w
        pltpu.sync_copy(idx_hbm.at[pl.ds(rbase, rows_per_w)], idx_v)

        def gather_desc(j, b):
            return pltpu.make_async_copy(
                tab_hbm.at[idx_v.at[j]], bufs[b], gs.at[b])

        def wback_desc(j, b):
            return pltpu.make_async_copy(
                bufs[b], out_hbm.at[pl.ds((rbase + j) * CB, CB)], ws.at[b])

        for b in range(nbuf):
            gather_desc(b, b).start()

        @pl.loop(0, rows_per_w, step=nbuf)
        def _grp(j0):
            for b in range(nbuf):
                gather_desc(j0 + b, b).wait()
                wback_desc(j0 + b, b).start()
            for b in range(nbuf):
                wback_desc(j0 + b, b).wait()

                @pl.when(j0 + nbuf < rows_per_w)
                def _():
                    gather_desc(j0 + nbuf + b, b).start()

    return k(idx2d, table)


def _mlp_body(xf_ref, xd_ref, xc_ref, xs_ref, w1_ref, b1_ref, w2_ref, c0_ref,
              w_ref, *, h, inv_s):
    xf = xf_ref[...]
    xd = xd_ref[...]
    xc = xc_ref[...]
    xs = xs_ref[...]

    def kstep(k, acc):
        a = (xf * w1_ref[k, 0] + xd * w1_ref[k, 1] + xc * w1_ref[k, 2]
             + xs * w1_ref[k, 3] + b1_ref[k, 0])
        return acc + w2_ref[k, 0] * jnp.tanh(a)

    acc = lax.fori_loop(0, h, kstep, jnp.zeros_like(xf))
    z = 0.25 * acc + 0.5 * c0_ref[0, 0]
    w_ref[...] = (0.5 * inv_s) * jnp.tanh(z) + (0.5 * inv_s)


def _edge_weights(xf, xd, xc, xs, w1h, b1h, w2c, c0, s):
    rtot = xf.shape[0]
    h = w1h.shape[0]
    grid = rtot // MLP_ROWS
    blk = pl.BlockSpec((MLP_ROWS, 128), lambda i: (i, 0))
    smem = pl.BlockSpec(memory_space=pltpu.SMEM)
    return pl.pallas_call(
        functools.partial(_mlp_body, h=h, inv_s=1.0 / s),
        grid=(grid,),
        in_specs=[blk, blk, blk, blk, smem, smem, smem, smem],
        out_specs=blk,
        out_shape=jax.ShapeDtypeStruct((rtot, 128), jnp.float32),
    )(xf, xd, xc, xs, w1h, b1h, w2c, c0)


def _reduce_body(w_ref, g_ref, o_ref, *, s):
    wg = g_ref[...] * w_ref[...]
    o_ref[...] = jnp.sum(wg.reshape(RED_ROWS // s, s, wg.shape[1]), axis=1)


def _weighted_reduce(wcol, gath, b, s, d):
    grid = (b * s) // RED_ROWS
    return pl.pallas_call(
        functools.partial(_reduce_body, s=s),
        grid=(grid,),
        in_specs=[
            pl.BlockSpec((RED_ROWS, 1), lambda i: (i, 0)),
            pl.BlockSpec((RED_ROWS, d), lambda i: (i, 0)),
        ],
        out_specs=pl.BlockSpec((RED_ROWS // s, d), lambda i: (i, 0)),
        out_shape=jax.ShapeDtypeStruct((b, d), jnp.float32),
    )(wcol, gath)


def kernel(nodes, neighs, freq_e, dist_e, feature_labels, spectral3, distance,
           features_table, weight1, bias1, weight2, bias2):
    b, s = neighs.shape
    n, d = features_table.shape
    h = weight1.shape[0]
    p = b * s

    unit = max(NW * CA, NW * CB * 8)
    ppad = ((p + unit - 1) // unit) * unit
    pad = ppad - p

    nodes_rep = jnp.broadcast_to(nodes[:, None], (b, s)).reshape(p)
    nodes_rep = jnp.pad(nodes_rep, (0, pad))
    neighs_flat = jnp.pad(neighs.reshape(p), (0, pad))

    spec, dcs = _small_gather(
        nodes_rep, neighs_flat,
        feature_labels.astype(jnp.int32), spectral3, distance)

    gath = _feat_gather(neighs_flat.reshape(ppad // CB, CB), features_table)

    rtot = ppad // 128
    xf = jnp.pad(freq_e.reshape(p), (0, pad)).reshape(rtot, 128)
    xd = jnp.pad(dist_e.reshape(p), (0, pad)).reshape(rtot, 128)
    xc = dcs.reshape(rtot, 128)
    xs = spec.reshape(rtot, 128)

    w1h = weight1 * 0.5
    b1h = bias1 * 0.5
    w2c = weight2.reshape(h, 1)
    c0 = (0.5 * jnp.sum(weight2) + bias2[0, 0]).reshape(1, 1)

    w4k = _edge_weights(xf, xd, xc, xs, w1h, b1h, w2c, c0, s)
    wcol = w4k.reshape(ppad)[:p].reshape(p, 1)

    return _weighted_reduce(wcol, gath, b, s, d)


# speedup vs baseline: 3.6324x; 1.0515x over previous
import dataclasses
import functools

import jax
import jax.numpy as jnp
from jax import lax
from jax.experimental import pallas as pl
from jax.experimental.pallas import tpu as pltpu
from jax.experimental.pallas import tpu_sc as plsc

NC, NS = 2, 16
NW = NC * NS
CA = 2048
CB = 128
MLP_ROWS = 32
RED_ROWS = 4000


def _sc_mesh():
    return plsc.VectorSubcoreMesh(
        core_axis_name="c", subcore_axis_name="s", num_cores=NC, num_subcores=NS
    )


def _sc_compiler_params():
    cp = pltpu.CompilerParams()
    if "needs_layout_passes" in pltpu.CompilerParams.__dataclass_fields__:
        cp = dataclasses.replace(cp, needs_layout_passes=False)
    return cp


def _small_gather(nodes_rep, neighs_flat, feature_labels, spectral3, distance):
    ppad = nodes_rep.shape[0]
    pw = ppad // NW
    n = feature_labels.shape[0]
    l = distance.shape[0]

    @functools.partial(
        pl.kernel,
        out_type=(
            jax.ShapeDtypeStruct((ppad,), jnp.float32),
            jax.ShapeDtypeStruct((ppad,), jnp.float32),
        ),
        mesh=_sc_mesh(),
        scratch_types=[
            pltpu.VMEM((n,), jnp.int32),
            pltpu.VMEM((n,), jnp.float32),
            pltpu.VMEM((l, l), jnp.float32),
            pltpu.VMEM((CA,), jnp.int32),
            pltpu.VMEM((CA,), jnp.int32),
            pltpu.VMEM((CA,), jnp.float32),
            pltpu.VMEM((CA,), jnp.float32),
        ],
        compiler_params=_sc_compiler_params(),
    )
    def k(nd_hbm, nb_hbm, fl_hbm, s3_hbm, di_hbm, spec_hbm, dcs_hbm,
          fl_v, s3_v, di_v, nd_c, nb_c, sp_c, dc_c):
        wid = lax.axis_index("s") * NC + lax.axis_index("c")
        base = wid * pw
        pltpu.sync_copy(fl_hbm, fl_v)
        pltpu.sync_copy(s3_hbm, s3_v)
        pltpu.sync_copy(di_hbm, di_v)

        @pl.loop(0, pw, step=CA)
        def _chunk(c0):
            pltpu.sync_copy(nd_hbm.at[pl.ds(base + c0, CA)], nd_c)
            pltpu.sync_copy(nb_hbm.at[pl.ds(base + c0, CA)], nb_c)

            @pl.loop(0, CA, step=16)
            def _vec(i):
                nd = nd_c[pl.ds(i, 16)]
                nb = nb_c[pl.ds(i, 16)]
                ls = plsc.load_gather(fl_v, [nd])
                ld = plsc.load_gather(fl_v, [nb])
                sp_c[pl.ds(i, 16)] = plsc.load_gather(s3_v, [nb])
                dc_c[pl.ds(i, 16)] = plsc.load_gather(di_v, [ls, ld])

            pltpu.sync_copy(sp_c, spec_hbm.at[pl.ds(base + c0, CA)])
            pltpu.sync_copy(dc_c, dcs_hbm.at[pl.ds(base + c0, CA)])

    return k(nodes_rep, neighs_flat, feature_labels, spectral3, distance)


def _feat_gather(idx2d, table):
    nrows, _ = idx2d.shape
    d = table.shape[1]
    rows_per_w = nrows // NW

    nbuf = 4

    @functools.partial(
        pl.kernel,
        out_type=jax.ShapeDtypeStruct((nrows * CB, d), jnp.float32),
        mesh=_sc_mesh(),
        scratch_types=[
            pltpu.VMEM((rows_per_w, CB), jnp.int32),
        ] + [pltpu.VMEM((CB, d), jnp.float32) for _ in range(nbuf)] + [
            pltpu.SemaphoreType.DMA((nbuf,)),
            pltpu.SemaphoreType.DMA((nbuf,)),
        ],
    )
    def k(idx_hbm, tab_hbm, out_hbm, idx_v, *rest):
        bufs = rest[:nbuf]
        gs, ws = rest[nbuf], rest[nbuf + 1]
        wid = lax.axis_index("s") * NC + lax.axis_index("c")
        rbase = wid * rows_per_w
        pltpu.sync_copy(idx_hbm.at[pl.ds(rbase, rows_per_w)], idx_v)

        def gather_desc(j, b):
            return pltpu.make_async_copy(
                tab_hbm.at[idx_v.at[j]], bufs[b], gs.at[b])

        def wback_desc(j, b):
            return pltpu.make_async_copy(
                bufs[b], out_hbm.at[pl.ds((rbase + j) * CB, CB)], ws.at[b])

        for b in range(nbuf):
            gather_desc(b, b).start()

        @pl.loop(0, rows_per_w, step=nbuf)
        def _grp(j0):
            for b in range(nbuf):
                gather_desc(j0 + b, b).wait()
                wback_desc(j0 + b, b).start()
            for b in range(nbuf):
                wback_desc(j0 + b, b).wait()

                @pl.when(j0 + nbuf < rows_per_w)
                def _():
                    gather_desc(j0 + nbuf + b, b).start()

    return k(idx2d, table)


def _mlp_body(xf_ref, xd_ref, xc_ref, xs_ref, w1_ref, b1_ref, w2_ref, c0_ref,
              w_ref, *, h, inv_s):
    xf = xf_ref[...]
    xd = xd_ref[...]
    xc = xc_ref[...]
    xs = xs_ref[...]

    unroll = 8

    def kstep(k0, acc):
        terms = []
        for u in range(unroll):
            k = k0 * unroll + u
            a = (xf * w1_ref[k, 0] + xd * w1_ref[k, 1] + xc * w1_ref[k, 2]
                 + xs * w1_ref[k, 3] + b1_ref[k, 0])
            terms.append(w2_ref[k, 0] * jnp.tanh(a))
        while len(terms) > 1:
            terms = [terms[i] + terms[i + 1] for i in range(0, len(terms), 2)]
        return acc + terms[0]

    acc = lax.fori_loop(0, h // unroll, kstep, jnp.zeros_like(xf))
    z = 0.25 * acc + 0.5 * c0_ref[0, 0]
    w_ref[...] = (0.5 * inv_s) * jnp.tanh(z) + (0.5 * inv_s)


def _edge_weights(xf, xd, xc, xs, w1h, b1h, w2c, c0, s):
    rtot = xf.shape[0]
    h = w1h.shape[0]
    grid = rtot // MLP_ROWS
    blk = pl.BlockSpec((MLP_ROWS, 128), lambda i: (i, 0))
    smem = pl.BlockSpec(memory_space=pltpu.SMEM)
    return pl.pallas_call(
        functools.partial(_mlp_body, h=h, inv_s=1.0 / s),
        grid=(grid,),
        in_specs=[blk, blk, blk, blk, smem, smem, smem, smem],
        out_specs=blk,
        out_shape=jax.ShapeDtypeStruct((rtot, 128), jnp.float32),
    )(xf, xd, xc, xs, w1h, b1h, w2c, c0)


def _reduce_body(w_ref, g_ref, o_ref, *, s):
    wg = g_ref[...] * w_ref[...]
    o_ref[...] = jnp.sum(wg.reshape(RED_ROWS // s, s, wg.shape[1]), axis=1)


def _weighted_reduce(wcol, gath, b, s, d):
    grid = (b * s) // RED_ROWS
    return pl.pallas_call(
        functools.partial(_reduce_body, s=s),
        grid=(grid,),
        in_specs=[
            pl.BlockSpec((RED_ROWS, 1), lambda i: (i, 0)),
            pl.BlockSpec((RED_ROWS, d), lambda i: (i, 0)),
        ],
        out_specs=pl.BlockSpec((RED_ROWS // s, d), lambda i: (i, 0)),
        out_shape=jax.ShapeDtypeStruct((b, d), jnp.float32),
    )(wcol, gath)


def kernel(nodes, neighs, freq_e, dist_e, feature_labels, spectral3, distance,
           features_table, weight1, bias1, weight2, bias2):
    b, s = neighs.shape
    n, d = features_table.shape
    h = weight1.shape[0]
    p = b * s

    unit = max(NW * CA, NW * CB * 8)
    ppad = ((p + unit - 1) // unit) * unit
    pad = ppad - p

    nodes_rep = jnp.broadcast_to(nodes[:, None], (b, s)).reshape(p)
    nodes_rep = jnp.pad(nodes_rep, (0, pad))
    neighs_flat = jnp.pad(neighs.reshape(p), (0, pad))

    spec, dcs = _small_gather(
        nodes_rep, neighs_flat,
        feature_labels.astype(jnp.int32), spectral3, distance)

    gath = _feat_gather(neighs_flat.reshape(ppad // CB, CB), features_table)

    rtot = ppad // 128
    xf = jnp.pad(freq_e.reshape(p), (0, pad)).reshape(rtot, 128)
    xd = jnp.pad(dist_e.reshape(p), (0, pad)).reshape(rtot, 128)
    xc = dcs.reshape(rtot, 128)
    xs = spec.reshape(rtot, 128)

    w1h = weight1 * 0.5
    b1h = bias1 * 0.5
    w2c = weight2.reshape(h, 1)
    c0 = (0.5 * jnp.sum(weight2) + bias2[0, 0]).reshape(1, 1)

    w4k = _edge_weights(xf, xd, xc, xs, w1h, b1h, w2c, c0, s)
    wcol = w4k.reshape(ppad)[:p].reshape(p, 1)

    return _weighted_reduce(wcol, gath, b, s, d)


# speedup vs baseline: 5.6643x; 1.5594x over previous
import dataclasses
import functools

import jax
import jax.numpy as jnp
from jax import lax
from jax.experimental import pallas as pl
from jax.experimental.pallas import tpu as pltpu
from jax.experimental.pallas import tpu_sc as plsc

NC, NS = 2, 16
NW = NC * NS
CA = 2048
CB = 128
MLP_ROWS = 2048
RED_ROWS = 4000


def _sc_mesh():
    return plsc.VectorSubcoreMesh(
        core_axis_name="c", subcore_axis_name="s", num_cores=NC, num_subcores=NS
    )


def _sc_compiler_params():
    cp = pltpu.CompilerParams()
    if "needs_layout_passes" in pltpu.CompilerParams.__dataclass_fields__:
        cp = dataclasses.replace(cp, needs_layout_passes=False)
    return cp


def _small_gather(nodes_rep, neighs_flat, feature_labels, spectral3, distance):
    ppad = nodes_rep.shape[0]
    pw = ppad // NW
    n = feature_labels.shape[0]
    l = distance.shape[0]

    @functools.partial(
        pl.kernel,
        out_type=(
            jax.ShapeDtypeStruct((ppad,), jnp.float32),
            jax.ShapeDtypeStruct((ppad,), jnp.float32),
        ),
        mesh=_sc_mesh(),
        scratch_types=[
            pltpu.VMEM((n,), jnp.int32),
            pltpu.VMEM((n,), jnp.float32),
            pltpu.VMEM((l, l), jnp.float32),
            pltpu.VMEM((CA,), jnp.int32),
            pltpu.VMEM((CA,), jnp.int32),
            pltpu.VMEM((CA,), jnp.float32),
            pltpu.VMEM((CA,), jnp.float32),
        ],
        compiler_params=_sc_compiler_params(),
    )
    def k(nd_hbm, nb_hbm, fl_hbm, s3_hbm, di_hbm, spec_hbm, dcs_hbm,
          fl_v, s3_v, di_v, nd_c, nb_c, sp_c, dc_c):
        wid = lax.axis_index("s") * NC + lax.axis_index("c")
        base = wid * pw
        pltpu.sync_copy(fl_hbm, fl_v)
        pltpu.sync_copy(s3_hbm, s3_v)
        pltpu.sync_copy(di_hbm, di_v)

        @pl.loop(0, pw, step=CA)
        def _chunk(c0):
            pltpu.sync_copy(nd_hbm.at[pl.ds(base + c0, CA)], nd_c)
            pltpu.sync_copy(nb_hbm.at[pl.ds(base + c0, CA)], nb_c)

            @pl.loop(0, CA, step=16)
            def _vec(i):
                nd = nd_c[pl.ds(i, 16)]
                nb = nb_c[pl.ds(i, 16)]
                ls = plsc.load_gather(fl_v, [nd])
                ld = plsc.load_gather(fl_v, [nb])
                sp_c[pl.ds(i, 16)] = plsc.load_gather(s3_v, [nb])
                dc_c[pl.ds(i, 16)] = plsc.load_gather(di_v, [ls, ld])

            pltpu.sync_copy(sp_c, spec_hbm.at[pl.ds(base + c0, CA)])
            pltpu.sync_copy(dc_c, dcs_hbm.at[pl.ds(base + c0, CA)])

    return k(nodes_rep, neighs_flat, feature_labels, spectral3, distance)


def _feat_gather(idx2d, table):
    nrows, _ = idx2d.shape
    d = table.shape[1]
    rows_per_w = nrows // NW

    nbuf = 4

    @functools.partial(
        pl.kernel,
        out_type=jax.ShapeDtypeStruct((nrows * CB, d), jnp.float32),
        mesh=_sc_mesh(),
        scratch_types=[
            pltpu.VMEM((rows_per_w, CB), jnp.int32),
        ] + [pltpu.VMEM((CB, d), jnp.float32) for _ in range(nbuf)] + [
            pltpu.SemaphoreType.DMA((nbuf,)),
            pltpu.SemaphoreType.DMA((nbuf,)),
        ],
    )
    def k(idx_hbm, tab_hbm, out_hbm, idx_v, *rest):
        bufs = rest[:nbuf]
        gs, ws = rest[nbuf], rest[nbuf + 1]
        wid = lax.axis_index("s") * NC + lax.axis_index("c")
        rbase = wid * rows_per_w
        pltpu.sync_copy(idx_hbm.at[pl.ds(rbase, rows_per_w)], idx_v)

        def gather_desc(j, b):
            return pltpu.make_async_copy(
                tab_hbm.at[idx_v.at[j]], bufs[b], gs.at[b])

        def wback_desc(j, b):
            return pltpu.make_async_copy(
                bufs[b], out_hbm.at[pl.ds((rbase + j) * CB, CB)], ws.at[b])

        for b in range(nbuf):
            gather_desc(b, b).start()

        @pl.loop(0, rows_per_w, step=nbuf)
        def _grp(j0):
            for b in range(nbuf):
                gather_desc(j0 + b, b).wait()
                wback_desc(j0 + b, b).start()
            for b in range(nbuf):
                wback_desc(j0 + b, b).wait()

                @pl.when(j0 + nbuf < rows_per_w)
                def _():
                    gather_desc(j0 + nbuf + b, b).start()

    return k(idx2d, table)


def _mlp_body(x_ref, w1_ref, b1_ref, w2_ref, c0_ref, w_ref, *, inv_s):
    a = jnp.dot(x_ref[...], w1_ref[...],
                preferred_element_type=jnp.float32) + b1_ref[...]
    t = jnp.tanh(a).astype(jnp.bfloat16)
    acc = jnp.dot(t, w2_ref[...], preferred_element_type=jnp.float32)
    z = 0.25 * acc + 0.5 * c0_ref[0, 0]
    w_ref[...] = (0.5 * inv_s) * jnp.tanh(z) + (0.5 * inv_s)


def _edge_weights(summary4, w1t, b1row, w2col, c0, s):
    ppad = summary4.shape[0]
    h = w1t.shape[1]
    grid = ppad // MLP_ROWS
    smem = pl.BlockSpec(memory_space=pltpu.SMEM)
    full = lambda shape: pl.BlockSpec(shape, lambda i: (0, 0))
    return pl.pallas_call(
        functools.partial(_mlp_body, inv_s=1.0 / s),
        grid=(grid,),
        in_specs=[pl.BlockSpec((MLP_ROWS, 4), lambda i: (i, 0)),
                  full((4, h)), full((1, h)), full((h, 1)), smem],
        out_specs=pl.BlockSpec((MLP_ROWS, 1), lambda i: (i, 0)),
        out_shape=jax.ShapeDtypeStruct((ppad, 1), jnp.float32),
    )(summary4, w1t, b1row, w2col, c0)


def _reduce_body(w_ref, g_ref, o_ref, *, s):
    wg = g_ref[...] * w_ref[...]
    o_ref[...] = jnp.sum(wg.reshape(RED_ROWS // s, s, wg.shape[1]), axis=1)


def _weighted_reduce(wcol, gath, b, s, d):
    grid = (b * s) // RED_ROWS
    return pl.pallas_call(
        functools.partial(_reduce_body, s=s),
        grid=(grid,),
        in_specs=[
            pl.BlockSpec((RED_ROWS, 1), lambda i: (i, 0)),
            pl.BlockSpec((RED_ROWS, d), lambda i: (i, 0)),
        ],
        out_specs=pl.BlockSpec((RED_ROWS // s, d), lambda i: (i, 0)),
        out_shape=jax.ShapeDtypeStruct((b, d), jnp.float32),
    )(wcol, gath)


def kernel(nodes, neighs, freq_e, dist_e, feature_labels, spectral3, distance,
           features_table, weight1, bias1, weight2, bias2):
    b, s = neighs.shape
    n, d = features_table.shape
    h = weight1.shape[0]
    p = b * s

    unit = max(NW * CA, NW * CB * 8)
    ppad = ((p + unit - 1) // unit) * unit
    pad = ppad - p

    nodes_rep = jnp.broadcast_to(nodes[:, None], (b, s)).reshape(p)
    nodes_rep = jnp.pad(nodes_rep, (0, pad))
    neighs_flat = jnp.pad(neighs.reshape(p), (0, pad))

    spec, dcs = _small_gather(
        nodes_rep, neighs_flat,
        feature_labels.astype(jnp.int32), spectral3, distance)

    gath = _feat_gather(neighs_flat.reshape(ppad // CB, CB), features_table)

    xf = jnp.pad(freq_e.reshape(p), (0, pad))
    xd = jnp.pad(dist_e.reshape(p), (0, pad))
    summary4 = jnp.stack([xf, xd, dcs, spec], axis=-1).astype(jnp.bfloat16)

    w1t = (weight1 * 0.5).T.astype(jnp.bfloat16)
    b1row = (bias1 * 0.5).reshape(1, h)
    w2col = weight2.reshape(h, 1).astype(jnp.bfloat16)
    c0 = (0.5 * jnp.sum(weight2) + bias2[0, 0]).reshape(1, 1)

    wcol = _edge_weights(summary4, w1t, b1row, w2col, c0, s)

    return _weighted_reduce(wcol, gath, b, s, d)
